# Initial kernel scaffold; baseline (speedup 1.0000x reference)
import jax
import jax.numpy as jnp
from jax.experimental import pallas as pl


def kernel(x, hyperedge_index, hyperedge_type, A, W_C, b_C):
    raise NotImplementedError("write your pallas kernel here")


# speedup vs baseline: 9.5954x; 9.5954x over previous
import functools
import jax
import jax.numpy as jnp
from jax import lax
from jax.experimental import pallas as pl
from jax.experimental.pallas import tpu as pltpu
from jax.experimental.pallas import tpu_sc as plsc

N = 10000
D = 128
O = 128
T = 4
S = 2
E = 160000

NC = 2
NS = 16
L = 16

C = 128
E_PAD = 163840
EPT_CNT = E_PAD // NS
EPT_MAIN = E_PAD // (NC * NS)
AGG_ROWS = 10112
CNT_SZ = 40960
YROWS = 9 * N


def _mm_body(x_ref, a_ref, y_ref):
    y_ref[...] = jnp.dot(x_ref[...], a_ref[0], preferred_element_type=jnp.float32)


def _tc_y(x, ab):
    return pl.pallas_call(
        _mm_body,
        grid=(9,),
        in_specs=[
            pl.BlockSpec((N, D), lambda k: (0, 0)),
            pl.BlockSpec((1, D, O), lambda k: (k, 0, 0)),
        ],
        out_specs=pl.BlockSpec((N, O), lambda k: (k, 0)),
        out_shape=jax.ShapeDtypeStruct((YROWS, O), jnp.float32),
    )(x, ab)


def _combine_body(yc_ref, agg_ref, b_ref, h_ref):
    h_ref[...] = yc_ref[...] + b_ref[...] + agg_ref[0, :, :] + agg_ref[1, :, :]


def _tc_combine(y, agg, b2):
    return pl.pallas_call(
        _combine_body,
        grid=(1,),
        in_specs=[
            pl.BlockSpec((N, O), lambda i: (8, 0)),
            pl.BlockSpec((NC, N, O), lambda i: (0, 0, 0)),
            pl.BlockSpec((1, O), lambda i: (0, 0)),
        ],
        out_specs=pl.BlockSpec((N, O), lambda i: (0, 0)),
        out_shape=jax.ShapeDtypeStruct((N, O), jnp.float32),
    )(y, agg, b2)


def _sc_body(y_hbm, s0_hbm, s1_hbm, dst_hbm, typ_hbm, out_hbm,
             zflat, y0b, y1b,
             s0b, s1b, dbuf, tbuf, g0b, g1b, cbuf, cvb, wbuf, onesb,
             agg_sp, cnt_sp, sem):
    cid = lax.axis_index("c")
    sid = lax.axis_index("s")

    @pl.loop(0, C)
    def _(r):
        for k in range(O // L):
            y0b[r, pl.ds(k * L, L)] = jnp.zeros((L,), jnp.float32)

    @pl.loop(0, 2560 // L)
    def _(i):
        zflat[pl.ds(i * L, L)] = jnp.zeros((L,), jnp.float32)

    for k in range(C // L):
        onesb[pl.ds(k * L, L)] = jnp.ones((L,), jnp.float32)

    rpt = AGG_ROWS // NS
    for j in range(4):
        pltpu.sync_copy(y0b, agg_sp.at[pl.ds(sid * rpt + j * C, C)])
    pltpu.sync_copy(y0b.at[pl.ds(0, rpt - 4 * C)],
                    agg_sp.at[pl.ds(sid * rpt + 4 * C, rpt - 4 * C)])
    pltpu.sync_copy(zflat, cnt_sp.at[pl.ds(sid * (CNT_SZ // NS), CNT_SZ // NS)])

    plsc.subcore_barrier()

    @pl.loop(0, EPT_CNT // C)
    def _(i):
        base = sid * EPT_CNT + i * C
        pltpu.sync_copy(dst_hbm.at[pl.ds(base, C)], dbuf)
        pltpu.sync_copy(typ_hbm.at[pl.ds(base, C)], tbuf)
        for k in range(C // L):
            sl = pl.ds(k * L, L)
            cbuf[sl] = dbuf[sl] * T + tbuf[sl]
        pltpu.sync_copy(onesb, cnt_sp.at[cbuf], add=True)

    plsc.subcore_barrier()

    @pl.loop(0, EPT_MAIN // C)
    def _(i):
        base = (cid * NS + sid) * EPT_MAIN + i * C
        pltpu.sync_copy(s0_hbm.at[pl.ds(base, C)], s0b)
        pltpu.sync_copy(s1_hbm.at[pl.ds(base, C)], s1b)
        pltpu.sync_copy(dst_hbm.at[pl.ds(base, C)], dbuf)
        pltpu.sync_copy(typ_hbm.at[pl.ds(base, C)], tbuf)
        for k in range(C // L):
            sl = pl.ds(k * L, L)
            ty = tbuf[sl]
            g0b[sl] = ty * (2 * N) + s0b[sl]
            g1b[sl] = ty * (2 * N) + N + s1b[sl]
            cbuf[sl] = dbuf[sl] * T + ty
        pltpu.sync_copy(cnt_sp.at[cbuf], cvb)
        for k in range(C // L):
            sl = pl.ds(k * L, L)
            wbuf[sl] = 1.0 / jnp.maximum(cvb[sl], 1.0)
        d0 = pltpu.async_copy(y_hbm.at[g0b], y0b, sem)
        d1 = pltpu.async_copy(y_hbm.at[g1b], y1b, sem)
        d0.wait()
        d1.wait()

        @pl.loop(0, C)
        def _(e):
            w = wbuf[pl.ds(e, L)][0]
            for k in range(O // L):
                sl = pl.ds(k * L, L)
                y0b[e, sl] = (y0b[e, sl] + y1b[e, sl]) * w

        pltpu.sync_copy(y0b, agg_sp.at[dbuf], add=True)

    plsc.subcore_barrier()

    rpt = AGG_ROWS // NS
    for j in range(4):
        r0 = sid * rpt + j * C
        pltpu.sync_copy(agg_sp.at[pl.ds(r0, C)], y0b)
        pltpu.sync_copy(y0b, out_hbm.at[cid, pl.ds(r0, C)])
    r0 = sid * rpt + 4 * C
    tail = rpt - 4 * C
    pltpu.sync_copy(agg_sp.at[pl.ds(r0, tail)], y0b.at[pl.ds(0, tail)])
    pltpu.sync_copy(y0b.at[pl.ds(0, tail)], out_hbm.at[cid, pl.ds(r0, tail)])


@functools.partial(
    pl.kernel,
    out_type=jax.ShapeDtypeStruct((NC, AGG_ROWS, O), jnp.float32),
    mesh=plsc.VectorSubcoreMesh(core_axis_name="c", subcore_axis_name="s"),
    scratch_types=[
        pltpu.VMEM((2560,), jnp.float32),
        pltpu.VMEM((C, O), jnp.float32),
        pltpu.VMEM((C, O), jnp.float32),
        pltpu.VMEM((C,), jnp.int32),
        pltpu.VMEM((C,), jnp.int32),
        pltpu.VMEM((C,), jnp.int32),
        pltpu.VMEM((C,), jnp.int32),
        pltpu.VMEM((C,), jnp.int32),
        pltpu.VMEM((C,), jnp.int32),
        pltpu.VMEM((C,), jnp.int32),
        pltpu.VMEM((C,), jnp.float32),
        pltpu.VMEM((C + L,), jnp.float32),
        pltpu.VMEM((C,), jnp.float32),
        pltpu.VMEM_SHARED((AGG_ROWS, O), jnp.float32),
        pltpu.VMEM_SHARED((CNT_SZ,), jnp.float32),
        pltpu.SemaphoreType.DMA,
    ],
)
def _sc_kernel(y, s0, s1, dst, typ, out, *scratch):
    _sc_body(y, s0, s1, dst, typ, out, *scratch)


@jax.jit
def kernel(x, hyperedge_index, hyperedge_type, A, W_C, b_C):
    hei = hyperedge_index.astype(jnp.int32)
    het = hyperedge_type.astype(jnp.int32)
    pad = E_PAD - E
    s0 = jnp.concatenate([hei[0, 0::2], jnp.zeros((pad,), jnp.int32)])
    s1 = jnp.concatenate([hei[0, 1::2], jnp.zeros((pad,), jnp.int32)])
    dst = jnp.concatenate([hei[1, 0::2], jnp.full((pad,), N, jnp.int32)])
    typ = jnp.concatenate([het, jnp.zeros((pad,), jnp.int32)])

    a8 = A.reshape(T, S, D, O).reshape(T * S, D, O)
    ab = jnp.concatenate([a8, W_C.T[None]], axis=0)

    y = _tc_y(x, ab)
    agg = _sc_kernel(y, s0, s1, dst, typ)
    return _tc_combine(y, agg, b_C.reshape(1, O))


# speedup vs baseline: 13.9613x; 1.4550x over previous
import functools
import jax
import jax.numpy as jnp
from jax import lax
from jax.experimental import pallas as pl
from jax.experimental.pallas import tpu as pltpu
from jax.experimental.pallas import tpu_sc as plsc

N = 10000
D = 128
O = 128
T = 4
S = 2
E = 160000

NC = 2
NS = 16
L = 16

C = 64
E_PAD = 163840
NCHUNKS = E_PAD // C
NCH1 = NCHUNKS // NS
NCH2 = NCHUNKS // (NC * NS)
AGG_ROWS = 10112
CNT_SZ = 40960
YROWS = 9 * N
RPT = AGG_ROWS // NS


def _mm_body(x_ref, a_ref, y_ref):
    y_ref[...] = jnp.dot(x_ref[...], a_ref[0], preferred_element_type=jnp.float32)


def _tc_y(x, ab):
    return pl.pallas_call(
        _mm_body,
        grid=(9,),
        in_specs=[
            pl.BlockSpec((N, D), lambda k: (0, 0)),
            pl.BlockSpec((1, D, O), lambda k: (k, 0, 0)),
        ],
        out_specs=pl.BlockSpec((N, O), lambda k: (k, 0)),
        out_shape=jax.ShapeDtypeStruct((YROWS, O), jnp.float32),
    )(x, ab)


def _combine_body(yc_ref, agg_ref, b_ref, h_ref):
    h_ref[...] = yc_ref[...] + b_ref[...] + agg_ref[0, :, :] + agg_ref[1, :, :]


def _tc_combine(y, agg, b2):
    return pl.pallas_call(
        _combine_body,
        grid=(1,),
        in_specs=[
            pl.BlockSpec((N, O), lambda i: (8, 0)),
            pl.BlockSpec((NC, N, O), lambda i: (0, 0, 0)),
            pl.BlockSpec((1, O), lambda i: (0, 0)),
        ],
        out_specs=pl.BlockSpec((N, O), lambda i: (0, 0)),
        out_shape=jax.ShapeDtypeStruct((N, O), jnp.float32),
    )(y, agg, b2)


def _sc_body(y_hbm, ed_hbm, out_hbm,
             zflat, y0b0, y0b1, y1b0, y1b1, eb0, eb1,
             g0b0, g0b1, g1b0, g1b1, c1b0, c1b1, c2b, cvb,
             wb0, wb1, onesb, agg_sp, cnt_sp, sem0, sem1):
    cid = lax.axis_index("c")
    sid = lax.axis_index("s")
    y0b = (y0b0, y0b1)
    y1b = (y1b0, y1b1)
    ebuf = (eb0, eb1)
    g0b = (g0b0, g0b1)
    g1b = (g1b0, g1b1)
    c1b = (c1b0, c1b1)
    wbuf = (wb0, wb1)
    sem = (sem0, sem1)

    @pl.loop(0, C)
    def _(r):
        for k in range(O // L):
            y0b0[r, pl.ds(k * L, L)] = jnp.zeros((L,), jnp.float32)

    @pl.loop(0, 2560 // L)
    def _(i):
        zflat[pl.ds(i * L, L)] = jnp.zeros((L,), jnp.float32)

    for k in range(C // L):
        onesb[pl.ds(k * L, L)] = jnp.ones((L,), jnp.float32)

    for j in range(9):
        pltpu.sync_copy(y0b0, agg_sp.at[pl.ds(sid * RPT + j * C, C)])
    pltpu.sync_copy(y0b0.at[pl.ds(0, RPT - 9 * C)],
                    agg_sp.at[pl.ds(sid * RPT + 9 * C, RPT - 9 * C)])
    pltpu.sync_copy(zflat, cnt_sp.at[pl.ds(sid * (CNT_SZ // NS), CNT_SZ // NS)])

    plsc.subcore_barrier()

    def _hist_prep(c, slot):
        pltpu.sync_copy(ed_hbm.at[sid * NCH1 + c], ebuf[slot])
        for k in range(C // L):
            sl = pl.ds(k * L, L)
            c1b[slot][sl] = ebuf[slot][2, sl] * T + ebuf[slot][3, sl]

    _hist_prep(0, 0)

    @pl.loop(0, NCH1, step=2)
    def _(i):
        for b in (0, 1):
            c = i + b

            @pl.when(c >= 1)
            def _():
                pltpu.make_async_copy(onesb, cnt_sp.at[c1b[1 - b]], sem[1 - b]).wait()

            @pl.when(c <= NCH1 - 2)
            def _():
                _hist_prep(c + 1, 1 - b)

            pltpu.async_copy(onesb, cnt_sp.at[c1b[b]], sem[b], add=True)

    pltpu.make_async_copy(onesb, cnt_sp.at[c1b[1]], sem[1]).wait()

    plsc.subcore_barrier()

    def _main_prep(c, slot):
        pltpu.sync_copy(ed_hbm.at[(cid * NS + sid) * NCH2 + c], ebuf[slot])
        for k in range(C // L):
            sl = pl.ds(k * L, L)
            ty = ebuf[slot][3, sl]
            g0b[slot][sl] = ty * (2 * N) + ebuf[slot][0, sl]
            g1b[slot][sl] = ty * (2 * N) + N + ebuf[slot][1, sl]
            c2b[sl] = ebuf[slot][2, sl] * T + ty
        pltpu.sync_copy(cnt_sp.at[c2b], cvb)
        for k in range(C // L):
            sl = pl.ds(k * L, L)
            wbuf[slot][sl] = 1.0 / jnp.maximum(cvb[sl], 1.0)
        pltpu.async_copy(y_hbm.at[g0b[slot]], y0b[slot], sem[slot])
        pltpu.async_copy(y_hbm.at[g1b[slot]], y1b[slot], sem[slot])

    _main_prep(0, 0)

    @pl.loop(0, NCH2, step=2)
    def _(i):
        for b in (0, 1):
            c = i + b

            @pl.when(c <= NCH2 - 2)
            def _():
                _main_prep(c + 1, 1 - b)

            pltpu.make_async_copy(y_hbm.at[g0b[b]], y0b[b], sem[b]).wait()
            pltpu.make_async_copy(y_hbm.at[g1b[b]], y1b[b], sem[b]).wait()

            @pl.loop(0, C)
            def _(e):
                w = wbuf[b][pl.ds(e, L)][0]
                for k in range(O // L):
                    sl = pl.ds(k * L, L)
                    y0b[b][e, sl] = (y0b[b][e, sl] + y1b[b][e, sl]) * w

            pltpu.sync_copy(y0b[b], agg_sp.at[ebuf[b].at[2]], add=True)

    plsc.subcore_barrier()

    for j in range(9):
        r0 = sid * RPT + j * C
        pltpu.sync_copy(agg_sp.at[pl.ds(r0, C)], y0b0)
        pltpu.sync_copy(y0b0, out_hbm.at[cid, pl.ds(r0, C)])
    r0 = sid * RPT + 9 * C
    tail = RPT - 9 * C
    pltpu.sync_copy(agg_sp.at[pl.ds(r0, tail)], y0b0.at[pl.ds(0, tail)])
    pltpu.sync_copy(y0b0.at[pl.ds(0, tail)], out_hbm.at[cid, pl.ds(r0, tail)])


@functools.partial(
    pl.kernel,
    out_type=jax.ShapeDtypeStruct((NC, AGG_ROWS, O), jnp.float32),
    mesh=plsc.VectorSubcoreMesh(core_axis_name="c", subcore_axis_name="s"),
    scratch_types=[
        pltpu.VMEM((2560,), jnp.float32),
        pltpu.VMEM((C, O), jnp.float32),
        pltpu.VMEM((C, O), jnp.float32),
        pltpu.VMEM((C, O), jnp.float32),
        pltpu.VMEM((C, O), jnp.float32),
        pltpu.VMEM((4, C), jnp.int32),
        pltpu.VMEM((4, C), jnp.int32),
        pltpu.VMEM((C,), jnp.int32),
        pltpu.VMEM((C,), jnp.int32),
        pltpu.VMEM((C,), jnp.int32),
        pltpu.VMEM((C,), jnp.int32),
        pltpu.VMEM((C,), jnp.int32),
        pltpu.VMEM((C,), jnp.int32),
        pltpu.VMEM((C,), jnp.int32),
        pltpu.VMEM((C,), jnp.float32),
        pltpu.VMEM((C + L,), jnp.float32),
        pltpu.VMEM((C + L,), jnp.float32),
        pltpu.VMEM((C,), jnp.float32),
        pltpu.VMEM_SHARED((AGG_ROWS, O), jnp.float32),
        pltpu.VMEM_SHARED((CNT_SZ,), jnp.float32),
        pltpu.SemaphoreType.DMA,
        pltpu.SemaphoreType.DMA,
    ],
)
def _sc_kernel(y, ed, out, *scratch):
    _sc_body(y, ed, out, *scratch)


@jax.jit
def kernel(x, hyperedge_index, hyperedge_type, A, W_C, b_C):
    hei = hyperedge_index.astype(jnp.int32)
    het = hyperedge_type.astype(jnp.int32)
    pad = E_PAD - E
    trash = N + (jnp.arange(pad, dtype=jnp.int32) % (AGG_ROWS - N))
    s0 = jnp.concatenate([hei[0, 0::2], jnp.zeros((pad,), jnp.int32)])
    s1 = jnp.concatenate([hei[0, 1::2], jnp.zeros((pad,), jnp.int32)])
    dst = jnp.concatenate([hei[1, 0::2], trash])
    typ = jnp.concatenate([het, jnp.zeros((pad,), jnp.int32)])
    ed = jnp.stack([s0, s1, dst, typ]).reshape(4, NCHUNKS, C).transpose(1, 0, 2)

    a8 = A.reshape(T, S, D, O).reshape(T * S, D, O)
    ab = jnp.concatenate([a8, W_C.T[None]], axis=0)

    y = _tc_y(x, ab)
    agg = _sc_kernel(y, ed)
    return _tc_combine(y, agg, b_C.reshape(1, O))


# speedup vs baseline: 15.4571x; 1.1071x over previous
import functools
import jax
import jax.numpy as jnp
from jax import lax
from jax.experimental import pallas as pl
from jax.experimental.pallas import tpu as pltpu
from jax.experimental.pallas import tpu_sc as plsc

N = 10000
D = 128
O = 128
T = 4
S = 2
E = 160000

NC = 2
NS = 16
L = 16

C = 64
NREAL = E // C
NCH1 = 160
NCH2 = 80
AGG_ROWS = 10112
CNT_SZ = 40960
YROWS = 9 * N
RPT = AGG_ROWS // NS


def _mm_body(x_ref, a_ref, y_ref):
    y_ref[...] = jnp.dot(x_ref[...], a_ref[0], preferred_element_type=jnp.float32)


def _tc_y(x, ab):
    return pl.pallas_call(
        _mm_body,
        grid=(9,),
        in_specs=[
            pl.BlockSpec((N, D), lambda k: (0, 0)),
            pl.BlockSpec((1, D, O), lambda k: (k, 0, 0)),
        ],
        out_specs=pl.BlockSpec((N, O), lambda k: (k, 0)),
        out_shape=jax.ShapeDtypeStruct((YROWS, O), jnp.float32),
    )(x, ab)


def _combine_body(yc_ref, agg_ref, b_ref, h_ref):
    h_ref[...] = yc_ref[...] + b_ref[...] + agg_ref[0, :, :] + agg_ref[1, :, :]


def _tc_combine(y, agg, b2):
    return pl.pallas_call(
        _combine_body,
        grid=(1,),
        in_specs=[
            pl.BlockSpec((N, O), lambda i: (8, 0)),
            pl.BlockSpec((NC, N, O), lambda i: (0, 0, 0)),
            pl.BlockSpec((1, O), lambda i: (0, 0)),
        ],
        out_specs=pl.BlockSpec((N, O), lambda i: (0, 0)),
        out_shape=jax.ShapeDtypeStruct((N, O), jnp.float32),
    )(y, agg, b2)


def _sc_body(y_hbm, hei0_hbm, hei1_hbm, typ_hbm, out_hbm,
             zflat, yp0, yp1, msgb, pb0, pb1, tr0, tr1,
             dp0, dp1, dp2, dp3, tb0, tb1, tb2, tb3,
             di0, di1, di2, di3, ti0, ti1,
             c1b0, c1b1, c1b2, c1b3, gp0, gp1,
             c2b, cvb, wb0, wb1, onesb, evens, halves,
             agg_sp, cnt_sp,
             isem0, isem1, isem2, isem3, ssem0, ssem1, ssem2, ssem3):
    cid = lax.axis_index("c")
    sid = lax.axis_index("s")
    ypair = (yp0, yp1)
    pbuf = (pb0, pb1)
    tyrep = (tr0, tr1)
    dp = (dp0, dp1, dp2, dp3)
    tb = (tb0, tb1, tb2, tb3)
    didx = (di0, di1, di2, di3)
    tyidx = (ti0, ti1)
    c1b = (c1b0, c1b1, c1b2, c1b3)
    gp = (gp0, gp1)
    wbuf = (wb0, wb1)
    isem = (isem0, isem1, isem2, isem3)
    ssem = (ssem0, ssem1, ssem2, ssem3)
    iota = lax.iota(jnp.int32, L)

    @pl.loop(0, C)
    def _(r):
        for k in range(O // L):
            msgb[r, pl.ds(k * L, L)] = jnp.zeros((L,), jnp.float32)

    @pl.loop(0, 2560 // L)
    def _(i):
        zflat[pl.ds(i * L, L)] = jnp.zeros((L,), jnp.float32)

    for k in range(C // L):
        onesb[pl.ds(k * L, L)] = jnp.ones((L,), jnp.float32)
        evens[pl.ds(k * L, L)] = iota * 2 + 32 * k
    for m in range(2 * C // L):
        halves[pl.ds(m * L, L)] = (iota >> 1) + 8 * m

    for j in range(9):
        pltpu.sync_copy(msgb, agg_sp.at[pl.ds(sid * RPT + j * C, C)])
    pltpu.sync_copy(msgb.at[pl.ds(0, RPT - 9 * C)],
                    agg_sp.at[pl.ds(sid * RPT + 9 * C, RPT - 9 * C)])
    pltpu.sync_copy(zflat, cnt_sp.at[pl.ds(sid * (CNT_SZ // NS), CNT_SZ // NS)])

    plsc.subcore_barrier()

    n1 = jnp.minimum(NREAL - sid * NCH1, NCH1)

    def _hist_idx_dma(c, s):
        base = (sid * NCH1 + c) * C
        for k in range(C // L):
            sl = pl.ds(k * L, L)
            didx[s][sl] = evens[sl] + 2 * base
        pltpu.async_copy(hei1_hbm.at[didx[s]], dp[s], isem[s])
        pltpu.async_copy(typ_hbm.at[pl.ds(base, C)], tb[s], isem[s])

    for s in range(4):
        _hist_idx_dma(s, s)

    @pl.loop(0, n1, step=4)
    def _(i):
        for b in range(4):
            c = i + b

            @pl.when(c >= 4)
            def _():
                pltpu.make_async_copy(onesb, cnt_sp.at[c1b[b]], ssem[b]).wait()

            pltpu.make_async_copy(hei1_hbm.at[didx[b]], dp[b], isem[b]).wait()
            pltpu.make_async_copy(typ_hbm.at[pl.ds(0, C)], tb[b], isem[b]).wait()
            for k in range(C // L):
                sl = pl.ds(k * L, L)
                c1b[b][sl] = dp[b][sl] * T + tb[b][sl]

            @pl.when(c + 4 < n1)
            def _():
                _hist_idx_dma(c + 4, b)

            pltpu.async_copy(onesb, cnt_sp.at[c1b[b]], ssem[b], add=True)

    for b in range(4):
        pltpu.make_async_copy(onesb, cnt_sp.at[c1b[b]], ssem[b]).wait()

    plsc.subcore_barrier()

    wid = cid * NS + sid
    n2 = jnp.minimum(jnp.maximum(NREAL - wid * NCH2, 0), NCH2)

    def _main_idx_dma(c, s):
        base = (wid * NCH2 + c) * C
        for k in range(C // L):
            sl = pl.ds(k * L, L)
            didx[s][sl] = evens[sl] + 2 * base
        for m in range(2 * C // L):
            sl = pl.ds(m * L, L)
            tyidx[s][sl] = halves[sl] + base
        pltpu.async_copy(hei0_hbm.at[pl.ds(2 * base, 2 * C)], pbuf[s], isem[s])
        pltpu.async_copy(typ_hbm.at[tyidx[s]], tyrep[s], isem[s])
        pltpu.async_copy(hei1_hbm.at[didx[s]], dp[s], isem[s])
        pltpu.async_copy(typ_hbm.at[pl.ds(base, C)], tb[s], isem[s])

    def _main_prep(c, s):
        pltpu.make_async_copy(hei0_hbm.at[pl.ds(0, 2 * C)], pbuf[s], isem[s]).wait()
        pltpu.make_async_copy(typ_hbm.at[tyidx[s]], tyrep[s], isem[s]).wait()
        pltpu.make_async_copy(hei1_hbm.at[didx[s]], dp[s], isem[s]).wait()
        pltpu.make_async_copy(typ_hbm.at[pl.ds(0, C)], tb[s], isem[s]).wait()
        oddn = (iota & 1) * N
        for m in range(2 * C // L):
            sl = pl.ds(m * L, L)
            gp[s][sl] = tyrep[s][sl] * (2 * N) + pbuf[s][sl] + oddn
        for k in range(C // L):
            sl = pl.ds(k * L, L)
            c2b[sl] = dp[s][sl] * T + tb[s][sl]
        pltpu.sync_copy(cnt_sp.at[c2b], cvb)
        for k in range(C // L):
            sl = pl.ds(k * L, L)
            wbuf[s][sl] = 1.0 / jnp.maximum(cvb[sl], 1.0)
        pltpu.async_copy(y_hbm.at[gp[s]], ypair[s], ssem[s])

    _main_idx_dma(0, 0)
    _main_prep(0, 0)

    @pl.when(1 < n2)
    def _():
        _main_idx_dma(1, 1)

    @pl.loop(0, n2, step=2)
    def _(i):
        for b in (0, 1):
            c = i + b

            @pl.when(c + 1 < n2)
            def _():
                _main_prep(c + 1, 1 - b)

            pltpu.make_async_copy(y_hbm.at[gp[b]], ypair[b], ssem[b]).wait()

            @pl.loop(0, C)
            def _(e):
                w = wbuf[b][pl.ds(e, L)][0]
                for k in range(O // L):
                    sl = pl.ds(k * L, L)
                    msgb[e, sl] = (ypair[b][2 * e, sl] + ypair[b][2 * e + 1, sl]) * w

            pltpu.sync_copy(msgb, agg_sp.at[dp[b]], add=True)

            @pl.when(c + 2 < n2)
            def _():
                _main_idx_dma(c + 2, b)

    plsc.subcore_barrier()

    for j in range(9):
        r0 = sid * RPT + j * C
        pltpu.sync_copy(agg_sp.at[pl.ds(r0, C)], msgb)
        pltpu.sync_copy(msgb, out_hbm.at[cid, pl.ds(r0, C)])
    r0 = sid * RPT + 9 * C
    tail = RPT - 9 * C
    pltpu.sync_copy(agg_sp.at[pl.ds(r0, tail)], msgb.at[pl.ds(0, tail)])
    pltpu.sync_copy(msgb.at[pl.ds(0, tail)], out_hbm.at[cid, pl.ds(r0, tail)])


@functools.partial(
    pl.kernel,
    out_type=jax.ShapeDtypeStruct((NC, AGG_ROWS, O), jnp.float32),
    mesh=plsc.VectorSubcoreMesh(core_axis_name="c", subcore_axis_name="s"),
    scratch_types=[
        pltpu.VMEM((2560,), jnp.float32),
        pltpu.VMEM((2 * C, O), jnp.float32),
        pltpu.VMEM((2 * C, O), jnp.float32),
        pltpu.VMEM((C, O), jnp.float32),
        pltpu.VMEM((2 * C,), jnp.int32),
        pltpu.VMEM((2 * C,), jnp.int32),
        pltpu.VMEM((2 * C,), jnp.int32),
        pltpu.VMEM((2 * C,), jnp.int32),
        pltpu.VMEM((C,), jnp.int32),
        pltpu.VMEM((C,), jnp.int32),
        pltpu.VMEM((C,), jnp.int32),
        pltpu.VMEM((C,), jnp.int32),
        pltpu.VMEM((C,), jnp.int32),
        pltpu.VMEM((C,), jnp.int32),
        pltpu.VMEM((C,), jnp.int32),
        pltpu.VMEM((C,), jnp.int32),
        pltpu.VMEM((C,), jnp.int32),
        pltpu.VMEM((C,), jnp.int32),
        pltpu.VMEM((C,), jnp.int32),
        pltpu.VMEM((C,), jnp.int32),
        pltpu.VMEM((2 * C,), jnp.int32),
        pltpu.VMEM((2 * C,), jnp.int32),
        pltpu.VMEM((C,), jnp.int32),
        pltpu.VMEM((C,), jnp.int32),
        pltpu.VMEM((C,), jnp.int32),
        pltpu.VMEM((C,), jnp.int32),
        pltpu.VMEM((2 * C,), jnp.int32),
        pltpu.VMEM((2 * C,), jnp.int32),
        pltpu.VMEM((C,), jnp.int32),
        pltpu.VMEM((C,), jnp.float32),
        pltpu.VMEM((C + L,), jnp.float32),
        pltpu.VMEM((C + L,), jnp.float32),
        pltpu.VMEM((C,), jnp.float32),
        pltpu.VMEM((C,), jnp.int32),
        pltpu.VMEM((2 * C,), jnp.int32),
        pltpu.VMEM_SHARED((AGG_ROWS, O), jnp.float32),
        pltpu.VMEM_SHARED((CNT_SZ,), jnp.float32),
        pltpu.SemaphoreType.DMA,
        pltpu.SemaphoreType.DMA,
        pltpu.SemaphoreType.DMA,
        pltpu.SemaphoreType.DMA,
        pltpu.SemaphoreType.DMA,
        pltpu.SemaphoreType.DMA,
        pltpu.SemaphoreType.DMA,
        pltpu.SemaphoreType.DMA,
    ],
)
def _sc_kernel(y, hei0, hei1, typ, out, *scratch):
    _sc_body(y, hei0, hei1, typ, out, *scratch)


@jax.jit
def kernel(x, hyperedge_index, hyperedge_type, A, W_C, b_C):
    hei = hyperedge_index.astype(jnp.int32)
    het = hyperedge_type.astype(jnp.int32)

    a8 = A.reshape(T, S, D, O).reshape(T * S, D, O)
    ab = jnp.concatenate([a8, W_C.T[None]], axis=0)

    y = _tc_y(x, ab)
    agg = _sc_kernel(y, hei[0], hei[1], het)
    return _tc_combine(y, agg, b_C.reshape(1, O))


# speedup vs baseline: 17.3817x; 1.1245x over previous
import functools
import jax
import jax.numpy as jnp
from jax import lax
from jax.experimental import pallas as pl
from jax.experimental.pallas import tpu as pltpu
from jax.experimental.pallas import tpu_sc as plsc

N = 10000
D = 128
O = 128
T = 4
S = 2
E = 160000

NC = 2
NS = 16
L = 16

C = 64
NREAL = E // C
NCH1 = 160
NCH2 = 80
AGG_ROWS = 10112
CNT_SZ = 40960
YROWS = 9 * N
RPT = AGG_ROWS // NS
EROWS = E // 128


def _mm_body(x_ref, a_ref, y_ref):
    y_ref[...] = jnp.dot(x_ref[...], a_ref[0], preferred_element_type=jnp.float32)


def _tc_y(x, ab):
    return pl.pallas_call(
        _mm_body,
        grid=(9,),
        in_specs=[
            pl.BlockSpec((N, D), lambda k: (0, 0)),
            pl.BlockSpec((1, D, O), lambda k: (k, 0, 0)),
        ],
        out_specs=pl.BlockSpec((N, O), lambda k: (k, 0)),
        out_shape=jax.ShapeDtypeStruct((YROWS, O), jnp.float32),
    )(x, ab)


def _prep_body(h0_ref, h1_ref, tp_ref, gp_ref, ci_ref, ds_ref):
    r = lax.broadcasted_iota(jnp.int32, (2 * D, D), 0)
    c = lax.broadcasted_iota(jnp.int32, (2 * D, D), 1)
    pe = (r == 2 * c).astype(jnp.float32)
    rq = lax.broadcasted_iota(jnp.int32, (D, 2 * D), 0)
    cq = lax.broadcasted_iota(jnp.int32, (D, 2 * D), 1)
    q = (rq == cq // 2).astype(jnp.float32)
    h0 = h0_ref[...].astype(jnp.float32)
    h1 = h1_ref[...].astype(jnp.float32)
    tp = tp_ref[...].astype(jnp.float32)
    dst = jnp.dot(h1, pe, preferred_element_type=jnp.float32,
                  precision=lax.Precision.HIGHEST)
    t2 = jnp.dot(tp, q, preferred_element_type=jnp.float32,
                 precision=lax.Precision.HIGHEST)
    odd = (lax.broadcasted_iota(jnp.int32, (1, 2 * D), 1) % 2).astype(jnp.float32)
    gp_ref[...] = (t2 * (2 * N) + h0 + odd * N).astype(jnp.int32)
    ci_ref[...] = (dst * T + tp).astype(jnp.int32)
    ds_ref[...] = dst.astype(jnp.int32)


def _tc_prep(h0, h1, tp):
    return pl.pallas_call(
        _prep_body,
        out_shape=[
            jax.ShapeDtypeStruct((EROWS, 2 * D), jnp.int32),
            jax.ShapeDtypeStruct((EROWS, D), jnp.int32),
            jax.ShapeDtypeStruct((EROWS, D), jnp.int32),
        ],
    )(h0, h1, tp)


def _combine_body(yc_ref, agg_ref, b_ref, h_ref):
    h_ref[...] = yc_ref[...] + b_ref[...] + agg_ref[0, :, :] + agg_ref[1, :, :]


def _tc_combine(y, agg, b2):
    return pl.pallas_call(
        _combine_body,
        grid=(1,),
        in_specs=[
            pl.BlockSpec((N, O), lambda i: (8, 0)),
            pl.BlockSpec((NC, N, O), lambda i: (0, 0, 0)),
            pl.BlockSpec((1, O), lambda i: (0, 0)),
        ],
        out_specs=pl.BlockSpec((N, O), lambda i: (0, 0)),
        out_shape=jax.ShapeDtypeStruct((N, O), jnp.float32),
    )(y, agg, b2)


def _sc_body(y_hbm, eda_hbm, edb_hbm, out_hbm,
             zflat, yp0, yp1, msgb, gpb0, gpb1, eb0, eb1,
             c1b0, c1b1, c1b2, c1b3,
             cvb, wb0, wb1, onesb, agg_sp, cnt_sp,
             isem0, isem1, isem2, isem3, ssem0, ssem1, ssem2, ssem3):
    cid = lax.axis_index("c")
    sid = lax.axis_index("s")
    ypair = (yp0, yp1)
    gpb = (gpb0, gpb1)
    ebuf = (eb0, eb1)
    c1b = (c1b0, c1b1, c1b2, c1b3)
    wbuf = (wb0, wb1)
    isem = (isem0, isem1, isem2, isem3)
    ssem = (ssem0, ssem1, ssem2, ssem3)

    @pl.loop(0, C)
    def _(r):
        for k in range(O // L):
            msgb[r, pl.ds(k * L, L)] = jnp.zeros((L,), jnp.float32)

    @pl.loop(0, 2560 // L)
    def _(i):
        zflat[pl.ds(i * L, L)] = jnp.zeros((L,), jnp.float32)

    for k in range(C // L):
        onesb[pl.ds(k * L, L)] = jnp.ones((L,), jnp.float32)

    for j in range(9):
        pltpu.sync_copy(msgb, agg_sp.at[pl.ds(sid * RPT + j * C, C)])
    pltpu.sync_copy(msgb.at[pl.ds(0, RPT - 9 * C)],
                    agg_sp.at[pl.ds(sid * RPT + 9 * C, RPT - 9 * C)])
    pltpu.sync_copy(zflat, cnt_sp.at[pl.ds(sid * (CNT_SZ // NS), CNT_SZ // NS)])

    plsc.subcore_barrier()

    n1 = jnp.minimum(NREAL - sid * NCH1, NCH1)

    def _hist_idx_dma(c, s):
        pltpu.async_copy(edb_hbm.at[sid * NCH1 + c, 0], c1b[s], isem[s])

    for s in range(4):
        _hist_idx_dma(s, s)

    @pl.loop(0, n1, step=4)
    def _(i):
        for b in range(4):
            c = i + b

            @pl.when(c >= 4)
            def _():
                pltpu.make_async_copy(onesb, cnt_sp.at[c1b[b]], ssem[b]).wait()

            pltpu.make_async_copy(edb_hbm.at[0, 0], c1b[b], isem[b]).wait()
            pltpu.async_copy(onesb, cnt_sp.at[c1b[b]], ssem[b], add=True)

            @pl.when(c + 4 < n1)
            def _():
                _hist_idx_dma(c + 4, b)

    for b in range(4):
        pltpu.make_async_copy(onesb, cnt_sp.at[c1b[b]], ssem[b]).wait()

    plsc.subcore_barrier()

    wid = cid * NS + sid
    n2 = jnp.minimum(jnp.maximum(NREAL - wid * NCH2, 0), NCH2)

    def _main_idx_dma(c, s):
        g = wid * NCH2 + c
        pltpu.async_copy(eda_hbm.at[g], gpb[s], isem[s])
        pltpu.async_copy(edb_hbm.at[g], ebuf[s], isem[s])

    def _main_prep(c, s):
        pltpu.make_async_copy(eda_hbm.at[0], gpb[s], isem[s]).wait()
        pltpu.make_async_copy(edb_hbm.at[0], ebuf[s], isem[s]).wait()
        pltpu.sync_copy(cnt_sp.at[ebuf[s].at[0]], cvb)
        for k in range(C // L):
            sl = pl.ds(k * L, L)
            wbuf[s][sl] = 1.0 / jnp.maximum(cvb[sl], 1.0)
        pltpu.async_copy(y_hbm.at[gpb[s]], ypair[s], ssem[s])

    _main_idx_dma(0, 0)
    _main_prep(0, 0)

    @pl.when(1 < n2)
    def _():
        _main_idx_dma(1, 1)

    @pl.loop(0, n2, step=2)
    def _(i):
        for b in (0, 1):
            c = i + b

            @pl.when(c + 1 < n2)
            def _():
                _main_prep(c + 1, 1 - b)

            pltpu.make_async_copy(y_hbm.at[gpb[b]], ypair[b], ssem[b]).wait()

            @pl.loop(0, C)
            def _(e):
                w = wbuf[b][pl.ds(e, L)][0]
                for k in range(O // L):
                    sl = pl.ds(k * L, L)
                    msgb[e, sl] = (ypair[b][2 * e, sl] + ypair[b][2 * e + 1, sl]) * w

            pltpu.sync_copy(msgb, agg_sp.at[ebuf[b].at[1]], add=True)

            @pl.when(c + 2 < n2)
            def _():
                _main_idx_dma(c + 2, b)

    plsc.subcore_barrier()

    for j in range(9):
        r0 = sid * RPT + j * C
        pltpu.sync_copy(agg_sp.at[pl.ds(r0, C)], msgb)
        pltpu.sync_copy(msgb, out_hbm.at[cid, pl.ds(r0, C)])
    r0 = sid * RPT + 9 * C
    tail = RPT - 9 * C
    pltpu.sync_copy(agg_sp.at[pl.ds(r0, tail)], msgb.at[pl.ds(0, tail)])
    pltpu.sync_copy(msgb.at[pl.ds(0, tail)], out_hbm.at[cid, pl.ds(r0, tail)])


@functools.partial(
    pl.kernel,
    out_type=jax.ShapeDtypeStruct((NC, AGG_ROWS, O), jnp.float32),
    mesh=plsc.VectorSubcoreMesh(core_axis_name="c", subcore_axis_name="s"),
    scratch_types=[
        pltpu.VMEM((2560,), jnp.float32),
        pltpu.VMEM((2 * C, O), jnp.float32),
        pltpu.VMEM((2 * C, O), jnp.float32),
        pltpu.VMEM((C, O), jnp.float32),
        pltpu.VMEM((2 * C,), jnp.int32),
        pltpu.VMEM((2 * C,), jnp.int32),
        pltpu.VMEM((2, C), jnp.int32),
        pltpu.VMEM((2, C), jnp.int32),
        pltpu.VMEM((C,), jnp.int32),
        pltpu.VMEM((C,), jnp.int32),
        pltpu.VMEM((C,), jnp.int32),
        pltpu.VMEM((C,), jnp.int32),
        pltpu.VMEM((C,), jnp.float32),
        pltpu.VMEM((C + L,), jnp.float32),
        pltpu.VMEM((C + L,), jnp.float32),
        pltpu.VMEM((C,), jnp.float32),
        pltpu.VMEM_SHARED((AGG_ROWS, O), jnp.float32),
        pltpu.VMEM_SHARED((CNT_SZ,), jnp.float32),
        pltpu.SemaphoreType.DMA,
        pltpu.SemaphoreType.DMA,
        pltpu.SemaphoreType.DMA,
        pltpu.SemaphoreType.DMA,
        pltpu.SemaphoreType.DMA,
        pltpu.SemaphoreType.DMA,
        pltpu.SemaphoreType.DMA,
        pltpu.SemaphoreType.DMA,
    ],
)
def _sc_kernel(y, eda, edb, out, *scratch):
    _sc_body(y, eda, edb, out, *scratch)


@jax.jit
def kernel(x, hyperedge_index, hyperedge_type, A, W_C, b_C):
    hei = hyperedge_index.astype(jnp.int32)
    het = hyperedge_type.astype(jnp.int32)

    a8 = A.reshape(T, S, D, O).reshape(T * S, D, O)
    ab = jnp.concatenate([a8, W_C.T[None]], axis=0)

    y = _tc_y(x, ab)
    gp, ci, ds = _tc_prep(hei[0].reshape(EROWS, 2 * D),
                          hei[1].reshape(EROWS, 2 * D),
                          het.reshape(EROWS, D))
    eda = gp.reshape(NREAL, 2 * C)
    edb = jnp.stack([ci.reshape(NREAL, C), ds.reshape(NREAL, C)], axis=1)
    agg = _sc_kernel(y, eda, edb)
    return _tc_combine(y, agg, b_C.reshape(1, O))


# speedup vs baseline: 20.0150x; 1.1515x over previous
import functools
import jax
import jax.numpy as jnp
from jax import lax
from jax.experimental import pallas as pl
from jax.experimental.pallas import tpu as pltpu
from jax.experimental.pallas import tpu_sc as plsc

N = 10000
D = 128
O = 128
T = 4
S = 2
E = 160000

NC = 2
NS = 16
L = 16

C = 64
NREAL = E // C
NCH2 = 80
EROWS = E // 128
NCH1 = 79
AGG_ROWS = 10112
CNT_SZ = 40960
YROWS = 9 * N
RPT = AGG_ROWS // NS


def _mm_body(x_ref, a_ref, y_ref):
    y_ref[...] = jnp.dot(x_ref[...], a_ref[0], preferred_element_type=jnp.float32)


def _tc_y(x, ab):
    return pl.pallas_call(
        _mm_body,
        grid=(9,),
        in_specs=[
            pl.BlockSpec((N, D), lambda k: (0, 0)),
            pl.BlockSpec((1, D, O), lambda k: (k, 0, 0)),
        ],
        out_specs=pl.BlockSpec((N, O), lambda k: (k, 0)),
        out_shape=jax.ShapeDtypeStruct((YROWS, O), jnp.float32),
    )(x, ab)


def _prep_body(h0_ref, h1_ref, tp_ref, gp_ref, ci_ref, ds_ref):
    r = lax.broadcasted_iota(jnp.int32, (2 * D, D), 0)
    c = lax.broadcasted_iota(jnp.int32, (2 * D, D), 1)
    pe = (r == 2 * c).astype(jnp.float32)
    rq = lax.broadcasted_iota(jnp.int32, (D, 2 * D), 0)
    cq = lax.broadcasted_iota(jnp.int32, (D, 2 * D), 1)
    q = (rq == cq // 2).astype(jnp.float32)
    h0 = h0_ref[...].astype(jnp.float32)
    h1 = h1_ref[...].astype(jnp.float32)
    tp = tp_ref[...].astype(jnp.float32)
    dst = jnp.dot(h1, pe, preferred_element_type=jnp.float32,
                  precision=lax.Precision.HIGHEST)
    t2 = jnp.dot(tp, q, preferred_element_type=jnp.float32,
                 precision=lax.Precision.HIGHEST)
    odd = (lax.broadcasted_iota(jnp.int32, (1, 2 * D), 1) % 2).astype(jnp.float32)
    gp_ref[...] = (t2 * (2 * N) + h0 + odd * N).astype(jnp.int32)
    ci_ref[...] = (dst * T + tp).astype(jnp.int32)
    ds_ref[...] = dst.astype(jnp.int32)


def _tc_prep(h0, h1, tp):
    return pl.pallas_call(
        _prep_body,
        out_shape=[
            jax.ShapeDtypeStruct((EROWS, 2 * D), jnp.int32),
            jax.ShapeDtypeStruct((EROWS, D), jnp.int32),
            jax.ShapeDtypeStruct((EROWS, D), jnp.int32),
        ],
    )(h0, h1, tp)


def _combine_body(yc_ref, agg_ref, b_ref, h_ref):
    h_ref[...] = yc_ref[...] + b_ref[...] + agg_ref[0, :, :] + agg_ref[1, :, :]


def _tc_combine(y, agg, b2):
    return pl.pallas_call(
        _combine_body,
        grid=(1,),
        in_specs=[
            pl.BlockSpec((N, O), lambda i: (8, 0)),
            pl.BlockSpec((NC, N, O), lambda i: (0, 0, 0)),
            pl.BlockSpec((1, O), lambda i: (0, 0)),
        ],
        out_specs=pl.BlockSpec((N, O), lambda i: (0, 0)),
        out_shape=jax.ShapeDtypeStruct((N, O), jnp.float32),
    )(y, agg, b2)


def _sc_body(y_hbm, eda_hbm, edb_hbm, edc_hbm, out_hbm,
             zflat, yp0, yp1, gpb0, gpb1, eb0, eb1,
             c1b0, c1b1, c1b2, c1b3, db0, db1,
             cvb0, cvb1, wbuf, onesb, agg_sp, cnt_sp,
             isem0, isem1, csem0, csem1, ysem0, ysem1, scsem0, scsem1):
    cid = lax.axis_index("c")
    sid = lax.axis_index("s")
    ypair = (yp0, yp1)
    gpb = (gpb0, gpb1)
    ebuf = (eb0, eb1)
    c1b = (c1b0, c1b1, c1b2, c1b3)
    dstb = (db0, db1)
    cvb = (cvb0, cvb1)
    isem = (isem0, isem1)
    csem = (csem0, csem1)
    ysem = (ysem0, ysem1)
    scsem = (scsem0, scsem1)
    hsem = (isem0, isem1, csem0, csem1)
    hssem = (ysem0, ysem1, scsem0, scsem1)

    @pl.loop(0, C)
    def _(r):
        for k in range(O // L):
            yp0[r, pl.ds(k * L, L)] = jnp.zeros((L,), jnp.float32)

    @pl.loop(0, 2560 // L)
    def _(i):
        zflat[pl.ds(i * L, L)] = jnp.zeros((L,), jnp.float32)

    for k in range(2 * C // L):
        onesb[pl.ds(k * L, L)] = jnp.ones((L,), jnp.float32)

    z64 = yp0.at[pl.ds(0, C)]
    for j in range(9):
        pltpu.sync_copy(z64, agg_sp.at[pl.ds(sid * RPT + j * C, C)])
    pltpu.sync_copy(yp0.at[pl.ds(0, RPT - 9 * C)],
                    agg_sp.at[pl.ds(sid * RPT + 9 * C, RPT - 9 * C)])
    pltpu.sync_copy(zflat, cnt_sp.at[pl.ds(sid * (CNT_SZ // NS), CNT_SZ // NS)])

    plsc.subcore_barrier()

    n1 = jnp.minimum(jnp.maximum(EROWS - sid * NCH1, 0), NCH1)

    def _hist_idx_dma(c, s):
        pltpu.async_copy(edc_hbm.at[sid * NCH1 + c], c1b[s], hsem[s])

    for s in range(4):
        _hist_idx_dma(s, s)

    @pl.loop(0, n1, step=4)
    def _(i):
        for b in range(4):
            c = i + b

            @pl.when(c < n1)
            def _():
                @pl.when(c >= 4)
                def _():
                    pltpu.make_async_copy(onesb, cnt_sp.at[c1b[b]], hssem[b]).wait()

                pltpu.make_async_copy(edc_hbm.at[0], c1b[b], hsem[b]).wait()
                pltpu.async_copy(onesb, cnt_sp.at[c1b[b]], hssem[b], add=True)

                @pl.when(c + 4 < n1)
                def _():
                    _hist_idx_dma(c + 4, b)

    for b in range(4):
        @pl.when(b < n1)
        def _():
            pltpu.make_async_copy(onesb, cnt_sp.at[c1b[b]], hssem[b]).wait()

    plsc.subcore_barrier()

    wid = cid * NS + sid
    n2 = jnp.minimum(jnp.maximum(NREAL - wid * NCH2, 0), NCH2)

    def _main_idx_dma(c, s):
        g = wid * NCH2 + c
        pltpu.async_copy(eda_hbm.at[g], gpb[s], isem[s])
        pltpu.async_copy(edb_hbm.at[g], ebuf[s], isem[s])

    def _main_prep(c, s):
        pltpu.make_async_copy(eda_hbm.at[0], gpb[s], isem[s]).wait()
        pltpu.make_async_copy(edb_hbm.at[0], ebuf[s], isem[s]).wait()
        for k in range(C // L):
            sl = pl.ds(k * L, L)
            dstb[s][sl] = ebuf[s][1, sl]
        pltpu.async_copy(cnt_sp.at[ebuf[s].at[0]], cvb[s], csem[s])
        pltpu.async_copy(y_hbm.at[gpb[s]], ypair[s], ysem[s])

    _main_idx_dma(0, 0)
    _main_prep(0, 0)

    @pl.when(1 < n2)
    def _():
        _main_idx_dma(1, 1)

    @pl.loop(0, n2, step=2)
    def _(i):
        for b in (0, 1):
            c = i + b

            @pl.when(c >= 1)
            def _():
                pltpu.make_async_copy(ypair[1 - b].at[pl.ds(0, C)],
                                      agg_sp.at[dstb[1 - b]], scsem[1 - b]).wait()

            @pl.when(c + 1 < n2)
            def _():
                _main_prep(c + 1, 1 - b)

            pltpu.make_async_copy(y_hbm.at[gpb[b]], ypair[b], ysem[b]).wait()
            pltpu.make_async_copy(cnt_sp.at[ebuf[b].at[0]], cvb[b], csem[b]).wait()
            for k in range(C // L):
                sl = pl.ds(k * L, L)
                wbuf[sl] = 1.0 / jnp.maximum(cvb[b][sl], 1.0)

            @pl.when(c + 2 < n2)
            def _():
                _main_idx_dma(c + 2, b)

            @pl.loop(0, C)
            def _(e):
                w = wbuf[pl.ds(e, L)][0]
                for k in range(O // L):
                    sl = pl.ds(k * L, L)
                    ypair[b][e, sl] = (ypair[b][2 * e, sl] + ypair[b][2 * e + 1, sl]) * w

            pltpu.async_copy(ypair[b].at[pl.ds(0, C)], agg_sp.at[dstb[b]],
                             scsem[b], add=True)

    pltpu.make_async_copy(ypair[1].at[pl.ds(0, C)], agg_sp.at[dstb[1]],
                          scsem[1]).wait()

    plsc.subcore_barrier()

    bounce = yp0.at[pl.ds(0, C)]
    for j in range(9):
        r0 = sid * RPT + j * C
        pltpu.sync_copy(agg_sp.at[pl.ds(r0, C)], bounce)
        pltpu.sync_copy(bounce, out_hbm.at[cid, pl.ds(r0, C)])
    r0 = sid * RPT + 9 * C
    tail = RPT - 9 * C
    pltpu.sync_copy(agg_sp.at[pl.ds(r0, tail)], yp0.at[pl.ds(0, tail)])
    pltpu.sync_copy(yp0.at[pl.ds(0, tail)], out_hbm.at[cid, pl.ds(r0, tail)])


@functools.partial(
    pl.kernel,
    out_type=jax.ShapeDtypeStruct((NC, AGG_ROWS, O), jnp.float32),
    mesh=plsc.VectorSubcoreMesh(core_axis_name="c", subcore_axis_name="s"),
    scratch_types=[
        pltpu.VMEM((2560,), jnp.float32),
        pltpu.VMEM((2 * C, O), jnp.float32),
        pltpu.VMEM((2 * C, O), jnp.float32),
        pltpu.VMEM((2 * C,), jnp.int32),
        pltpu.VMEM((2 * C,), jnp.int32),
        pltpu.VMEM((2, C), jnp.int32),
        pltpu.VMEM((2, C), jnp.int32),
        pltpu.VMEM((2 * C,), jnp.int32),
        pltpu.VMEM((2 * C,), jnp.int32),
        pltpu.VMEM((2 * C,), jnp.int32),
        pltpu.VMEM((2 * C,), jnp.int32),
        pltpu.VMEM((C,), jnp.int32),
        pltpu.VMEM((C,), jnp.int32),
        pltpu.VMEM((C,), jnp.float32),
        pltpu.VMEM((C,), jnp.float32),
        pltpu.VMEM((C + L,), jnp.float32),
        pltpu.VMEM((2 * C,), jnp.float32),
        pltpu.VMEM_SHARED((AGG_ROWS, O), jnp.float32),
        pltpu.VMEM_SHARED((CNT_SZ,), jnp.float32),
        pltpu.SemaphoreType.DMA,
        pltpu.SemaphoreType.DMA,
        pltpu.SemaphoreType.DMA,
        pltpu.SemaphoreType.DMA,
        pltpu.SemaphoreType.DMA,
        pltpu.SemaphoreType.DMA,
        pltpu.SemaphoreType.DMA,
        pltpu.SemaphoreType.DMA,
    ],
)
def _sc_kernel(y, eda, edb, edc, out, *scratch):
    _sc_body(y, eda, edb, edc, out, *scratch)


@jax.jit
def kernel(x, hyperedge_index, hyperedge_type, A, W_C, b_C):
    hei = hyperedge_index.astype(jnp.int32)
    het = hyperedge_type.astype(jnp.int32)

    a8 = A.reshape(T, S, D, O).reshape(T * S, D, O)
    ab = jnp.concatenate([a8, W_C.T[None]], axis=0)

    y = _tc_y(x, ab)
    gp, ci, ds = _tc_prep(hei[0].reshape(EROWS, 2 * D),
                          hei[1].reshape(EROWS, 2 * D),
                          het.reshape(EROWS, D))
    eda = gp.reshape(NREAL, 2 * C)
    edb = jnp.stack([ci.reshape(NREAL, C), ds.reshape(NREAL, C)], axis=1)
    agg = _sc_kernel(y, eda, edb, ci)
    return _tc_combine(y, agg, b_C.reshape(1, O))


# speedup vs baseline: 20.4445x; 1.0215x over previous
import functools
import jax
import jax.numpy as jnp
from jax import lax
from jax.experimental import pallas as pl
from jax.experimental.pallas import tpu as pltpu
from jax.experimental.pallas import tpu_sc as plsc

N = 10000
D = 128
O = 128
T = 4
S = 2
E = 160000

NC = 2
NS = 16
L = 16

C = 64
NREAL = E // C
NCH2 = 80
EROWS = E // 128
NCH1 = 79
AGG_ROWS = 10112
CNT_SZ = 40960
YROWS = 9 * N
RPT = AGG_ROWS // NS


def _mm_body(x_ref, a_ref, y_ref):
    y_ref[...] = jnp.dot(x_ref[...], a_ref[0], preferred_element_type=jnp.float32)


def _tc_y(x, ab):
    return pl.pallas_call(
        _mm_body,
        grid=(9,),
        in_specs=[
            pl.BlockSpec((N, D), lambda k: (0, 0)),
            pl.BlockSpec((1, D, O), lambda k: (k, 0, 0)),
        ],
        out_specs=pl.BlockSpec((N, O), lambda k: (k, 0)),
        out_shape=jax.ShapeDtypeStruct((YROWS, O), jnp.float32),
    )(x, ab)


def _prep_body(h_ref, tp_ref, gp_ref, ci_ref, ds_ref):
    r = lax.broadcasted_iota(jnp.int32, (2 * D, D), 0)
    c = lax.broadcasted_iota(jnp.int32, (2 * D, D), 1)
    pe = (r == 2 * c).astype(jnp.float32)
    rq = lax.broadcasted_iota(jnp.int32, (D, 2 * D), 0)
    cq = lax.broadcasted_iota(jnp.int32, (D, 2 * D), 1)
    q = (rq == cq // 2).astype(jnp.float32)
    h0 = h_ref[0].astype(jnp.float32)
    h1 = h_ref[1].astype(jnp.float32)
    tp = tp_ref[...].astype(jnp.float32)
    dst = jnp.dot(h1, pe, preferred_element_type=jnp.float32,
                  precision=lax.Precision.HIGHEST)
    t2 = jnp.dot(tp, q, preferred_element_type=jnp.float32,
                 precision=lax.Precision.HIGHEST)
    odd = (lax.broadcasted_iota(jnp.int32, (1, 2 * D), 1) % 2).astype(jnp.float32)
    gp_ref[...] = (t2 * (2 * N) + h0 + odd * N).astype(jnp.int32)
    ci_ref[...] = (dst * T + tp).astype(jnp.int32)
    ds_ref[...] = dst.astype(jnp.int32)


def _tc_prep(h, tp):
    return pl.pallas_call(
        _prep_body,
        out_shape=[
            jax.ShapeDtypeStruct((EROWS, 2 * D), jnp.int32),
            jax.ShapeDtypeStruct((EROWS, D), jnp.int32),
            jax.ShapeDtypeStruct((EROWS, D), jnp.int32),
        ],
    )(h, tp)


def _combine_body(yc_ref, agg_ref, b_ref, h_ref):
    h_ref[...] = yc_ref[...] + b_ref[...] + agg_ref[0, :, :] + agg_ref[1, :, :]


def _tc_combine(y, agg, b2):
    return pl.pallas_call(
        _combine_body,
        grid=(1,),
        in_specs=[
            pl.BlockSpec((N, O), lambda i: (8, 0)),
            pl.BlockSpec((NC, N, O), lambda i: (0, 0, 0)),
            pl.BlockSpec((1, O), lambda i: (0, 0)),
        ],
        out_specs=pl.BlockSpec((N, O), lambda i: (0, 0)),
        out_shape=jax.ShapeDtypeStruct((N, O), jnp.float32),
    )(y, agg, b2)


def _sc_body(y_hbm, eda_hbm, edb_hbm, edc_hbm, out_hbm,
             zflat, yp0, yp1, gpb0, gpb1, eb0, eb1,
             c1b0, c1b1, c1b2, c1b3, db0, db1,
             cvb0, cvb1, wbuf, onesb, agg_sp, cnt_sp,
             isem0, isem1, csem0, csem1, ysem0, ysem1, scsem0, scsem1):
    cid = lax.axis_index("c")
    sid = lax.axis_index("s")
    ypair = (yp0, yp1)
    gpb = (gpb0, gpb1)
    ebuf = (eb0, eb1)
    c1b = (c1b0, c1b1, c1b2, c1b3)
    dstb = (db0, db1)
    cvb = (cvb0, cvb1)
    isem = (isem0, isem1)
    csem = (csem0, csem1)
    ysem = (ysem0, ysem1)
    scsem = (scsem0, scsem1)
    hsem = (isem0, isem1, csem0, csem1)
    hssem = (ysem0, ysem1, scsem0, scsem1)

    @pl.loop(0, C)
    def _(r):
        for k in range(O // L):
            yp0[r, pl.ds(k * L, L)] = jnp.zeros((L,), jnp.float32)

    @pl.loop(0, 2560 // L)
    def _(i):
        zflat[pl.ds(i * L, L)] = jnp.zeros((L,), jnp.float32)

    for k in range(2 * C // L):
        onesb[pl.ds(k * L, L)] = jnp.ones((L,), jnp.float32)

    z64 = yp0.at[pl.ds(0, C)]
    for j in range(9):
        pltpu.sync_copy(z64, agg_sp.at[pl.ds(sid * RPT + j * C, C)])
    pltpu.sync_copy(yp0.at[pl.ds(0, RPT - 9 * C)],
                    agg_sp.at[pl.ds(sid * RPT + 9 * C, RPT - 9 * C)])
    pltpu.sync_copy(zflat, cnt_sp.at[pl.ds(sid * (CNT_SZ // NS), CNT_SZ // NS)])

    plsc.subcore_barrier()

    n1 = jnp.minimum(jnp.maximum(EROWS - sid * NCH1, 0), NCH1)

    def _hist_idx_dma(c, s):
        pltpu.async_copy(edc_hbm.at[sid * NCH1 + c], c1b[s], hsem[s])

    for s in range(4):
        _hist_idx_dma(s, s)

    @pl.loop(0, n1, step=4)
    def _(i):
        for b in range(4):
            c = i + b

            @pl.when(c < n1)
            def _():
                @pl.when(c >= 4)
                def _():
                    pltpu.make_async_copy(onesb, cnt_sp.at[c1b[b]], hssem[b]).wait()

                pltpu.make_async_copy(edc_hbm.at[0], c1b[b], hsem[b]).wait()
                pltpu.async_copy(onesb, cnt_sp.at[c1b[b]], hssem[b], add=True)

                @pl.when(c + 4 < n1)
                def _():
                    _hist_idx_dma(c + 4, b)

    for b in range(4):
        @pl.when(b < n1)
        def _():
            pltpu.make_async_copy(onesb, cnt_sp.at[c1b[b]], hssem[b]).wait()

    plsc.subcore_barrier()

    wid = cid * NS + sid
    n2 = jnp.minimum(jnp.maximum(NREAL - wid * NCH2, 0), NCH2)

    def _main_idx_dma(c, s):
        g = wid * NCH2 + c
        pltpu.async_copy(eda_hbm.at[g], gpb[s], isem[s])
        pltpu.async_copy(edb_hbm.at[g], ebuf[s], isem[s])

    def _main_prep(c, s):
        pltpu.make_async_copy(eda_hbm.at[0], gpb[s], isem[s]).wait()
        pltpu.make_async_copy(edb_hbm.at[0], ebuf[s], isem[s]).wait()
        for k in range(C // L):
            sl = pl.ds(k * L, L)
            dstb[s][sl] = ebuf[s][1, sl]
        pltpu.async_copy(cnt_sp.at[ebuf[s].at[0]], cvb[s], csem[s])
        pltpu.async_copy(y_hbm.at[gpb[s]], ypair[s], ysem[s])

    _main_idx_dma(0, 0)
    _main_prep(0, 0)

    @pl.when(1 < n2)
    def _():
        _main_idx_dma(1, 1)

    @pl.loop(0, n2, step=2)
    def _(i):
        for b in (0, 1):
            c = i + b

            @pl.when(c >= 1)
            def _():
                pltpu.make_async_copy(ypair[1 - b].at[pl.ds(0, C)],
                                      agg_sp.at[dstb[1 - b]], scsem[1 - b]).wait()

            @pl.when(c + 1 < n2)
            def _():
                _main_prep(c + 1, 1 - b)

            pltpu.make_async_copy(y_hbm.at[gpb[b]], ypair[b], ysem[b]).wait()
            pltpu.make_async_copy(cnt_sp.at[ebuf[b].at[0]], cvb[b], csem[b]).wait()
            for k in range(C // L):
                sl = pl.ds(k * L, L)
                wbuf[sl] = 1.0 / jnp.maximum(cvb[b][sl], 1.0)

            @pl.when(c + 2 < n2)
            def _():
                _main_idx_dma(c + 2, b)

            @pl.loop(0, C, unroll=2)
            def _(e):
                w = wbuf[pl.ds(e, L)][0]
                for k in range(O // L):
                    sl = pl.ds(k * L, L)
                    ypair[b][e, sl] = (ypair[b][2 * e, sl] + ypair[b][2 * e + 1, sl]) * w

            pltpu.async_copy(ypair[b].at[pl.ds(0, C)], agg_sp.at[dstb[b]],
                             scsem[b], add=True)

    pltpu.make_async_copy(ypair[1].at[pl.ds(0, C)], agg_sp.at[dstb[1]],
                          scsem[1]).wait()

    plsc.subcore_barrier()

    bounce = yp0.at[pl.ds(0, C)]
    for j in range(9):
        r0 = sid * RPT + j * C
        pltpu.sync_copy(agg_sp.at[pl.ds(r0, C)], bounce)
        pltpu.sync_copy(bounce, out_hbm.at[cid, pl.ds(r0, C)])
    r0 = sid * RPT + 9 * C
    tail = RPT - 9 * C
    pltpu.sync_copy(agg_sp.at[pl.ds(r0, tail)], yp0.at[pl.ds(0, tail)])
    pltpu.sync_copy(yp0.at[pl.ds(0, tail)], out_hbm.at[cid, pl.ds(r0, tail)])


@functools.partial(
    pl.kernel,
    out_type=jax.ShapeDtypeStruct((NC, AGG_ROWS, O), jnp.float32),
    mesh=plsc.VectorSubcoreMesh(core_axis_name="c", subcore_axis_name="s"),
    scratch_types=[
        pltpu.VMEM((2560,), jnp.float32),
        pltpu.VMEM((2 * C, O), jnp.float32),
        pltpu.VMEM((2 * C, O), jnp.float32),
        pltpu.VMEM((2 * C,), jnp.int32),
        pltpu.VMEM((2 * C,), jnp.int32),
        pltpu.VMEM((2, C), jnp.int32),
        pltpu.VMEM((2, C), jnp.int32),
        pltpu.VMEM((2 * C,), jnp.int32),
        pltpu.VMEM((2 * C,), jnp.int32),
        pltpu.VMEM((2 * C,), jnp.int32),
        pltpu.VMEM((2 * C,), jnp.int32),
        pltpu.VMEM((C,), jnp.int32),
        pltpu.VMEM((C,), jnp.int32),
        pltpu.VMEM((C,), jnp.float32),
        pltpu.VMEM((C,), jnp.float32),
        pltpu.VMEM((C + L,), jnp.float32),
        pltpu.VMEM((2 * C,), jnp.float32),
        pltpu.VMEM_SHARED((AGG_ROWS, O), jnp.float32),
        pltpu.VMEM_SHARED((CNT_SZ,), jnp.float32),
        pltpu.SemaphoreType.DMA,
        pltpu.SemaphoreType.DMA,
        pltpu.SemaphoreType.DMA,
        pltpu.SemaphoreType.DMA,
        pltpu.SemaphoreType.DMA,
        pltpu.SemaphoreType.DMA,
        pltpu.SemaphoreType.DMA,
        pltpu.SemaphoreType.DMA,
    ],
)
def _sc_kernel(y, eda, edb, edc, out, *scratch):
    _sc_body(y, eda, edb, edc, out, *scratch)


@jax.jit
def kernel(x, hyperedge_index, hyperedge_type, A, W_C, b_C):
    hei = hyperedge_index.astype(jnp.int32)
    het = hyperedge_type.astype(jnp.int32)

    a8 = A.reshape(T, S, D, O).reshape(T * S, D, O)
    ab = jnp.concatenate([a8, W_C.T[None]], axis=0)

    y = _tc_y(x, ab)
    gp, ci, ds = _tc_prep(hei.reshape(2, EROWS, 2 * D), het.reshape(EROWS, D))
    eda = gp.reshape(NREAL, 2 * C)
    edb = jnp.stack([ci.reshape(NREAL, C), ds.reshape(NREAL, C)], axis=1)
    agg = _sc_kernel(y, eda, edb, ci)
    return _tc_combine(y, agg, b_C.reshape(1, O))


# speedup vs baseline: 20.5541x; 1.0054x over previous
import functools
import jax
import jax.numpy as jnp
from jax import lax
from jax.experimental import pallas as pl
from jax.experimental.pallas import tpu as pltpu
from jax.experimental.pallas import tpu_sc as plsc

N = 10000
D = 128
O = 128
T = 4
S = 2
E = 160000

NC = 2
NS = 16
L = 16

C = 64
NREAL = E // C
NCH2 = 80
EROWS = E // 128
NCH1 = 79
AGG_ROWS = 10112
CNT_SZ = 40960
YROWS = 8 * N
RPT = AGG_ROWS // NS


def _mm_body(x_ref, a_ref, y_ref):
    y_ref[...] = jnp.dot(x_ref[...], a_ref[0], preferred_element_type=jnp.float32)


def _tc_y(x, ab):
    return pl.pallas_call(
        _mm_body,
        grid=(8,),
        in_specs=[
            pl.BlockSpec((N, D), lambda k: (0, 0)),
            pl.BlockSpec((1, D, O), lambda k: (k, 0, 0)),
        ],
        out_specs=pl.BlockSpec((N, O), lambda k: (k, 0)),
        out_shape=jax.ShapeDtypeStruct((YROWS, O), jnp.float32),
    )(x, ab)


def _prep_body(h_ref, tp_ref, gp_ref, ci_ref, ds_ref):
    r = lax.broadcasted_iota(jnp.int32, (2 * D, D), 0)
    c = lax.broadcasted_iota(jnp.int32, (2 * D, D), 1)
    pe = (r == 2 * c).astype(jnp.float32)
    rq = lax.broadcasted_iota(jnp.int32, (D, 2 * D), 0)
    cq = lax.broadcasted_iota(jnp.int32, (D, 2 * D), 1)
    q = (rq == cq // 2).astype(jnp.float32)
    h0 = h_ref[0].astype(jnp.float32)
    h1 = h_ref[1].astype(jnp.float32)
    tp = tp_ref[...].astype(jnp.float32)
    dst = jnp.dot(h1, pe, preferred_element_type=jnp.float32,
                  precision=lax.Precision.HIGHEST)
    t2 = jnp.dot(tp, q, preferred_element_type=jnp.float32,
                 precision=lax.Precision.HIGHEST)
    odd = (lax.broadcasted_iota(jnp.int32, (1, 2 * D), 1) % 2).astype(jnp.float32)
    gp_ref[...] = (t2 * (2 * N) + h0 + odd * N).astype(jnp.int32)
    ci_ref[...] = (dst * T + tp).astype(jnp.int32)
    ds_ref[...] = dst.astype(jnp.int32)


def _tc_prep(h, tp):
    return pl.pallas_call(
        _prep_body,
        out_shape=[
            jax.ShapeDtypeStruct((EROWS, 2 * D), jnp.int32),
            jax.ShapeDtypeStruct((EROWS, D), jnp.int32),
            jax.ShapeDtypeStruct((EROWS, D), jnp.int32),
        ],
    )(h, tp)


def _combine_body(x_ref, wt_ref, agg_ref, b_ref, h_ref):
    h_ref[...] = (
        jnp.dot(x_ref[...], wt_ref[...], preferred_element_type=jnp.float32)
        + b_ref[...] + agg_ref[0, :, :] + agg_ref[1, :, :]
    )


def _tc_combine(x, wt, agg, b2):
    return pl.pallas_call(
        _combine_body,
        grid=(1,),
        in_specs=[
            pl.BlockSpec((N, D), lambda i: (0, 0)),
            pl.BlockSpec((D, O), lambda i: (0, 0)),
            pl.BlockSpec((NC, N, O), lambda i: (0, 0, 0)),
            pl.BlockSpec((1, O), lambda i: (0, 0)),
        ],
        out_specs=pl.BlockSpec((N, O), lambda i: (0, 0)),
        out_shape=jax.ShapeDtypeStruct((N, O), jnp.float32),
    )(x, wt, agg, b2)


def _sc_body(y_hbm, eda_hbm, edb_hbm, edc_hbm, out_hbm,
             zflat, yp0, yp1, gpb0, gpb1, eb0, eb1,
             c1b0, c1b1, c1b2, c1b3, db0, db1,
             cvb0, cvb1, wbuf, onesb, agg_sp, cnt_sp,
             isem0, isem1, csem0, csem1, ysem0, ysem1, scsem0, scsem1):
    cid = lax.axis_index("c")
    sid = lax.axis_index("s")
    ypair = (yp0, yp1)
    gpb = (gpb0, gpb1)
    ebuf = (eb0, eb1)
    c1b = (c1b0, c1b1, c1b2, c1b3)
    dstb = (db0, db1)
    cvb = (cvb0, cvb1)
    isem = (isem0, isem1)
    csem = (csem0, csem1)
    ysem = (ysem0, ysem1)
    scsem = (scsem0, scsem1)
    hsem = (isem0, isem1, csem0, csem1)
    hssem = (ysem0, ysem1, scsem0, scsem1)

    @pl.loop(0, C)
    def _(r):
        for k in range(O // L):
            yp0[r, pl.ds(k * L, L)] = jnp.zeros((L,), jnp.float32)

    @pl.loop(0, 2560 // L)
    def _(i):
        zflat[pl.ds(i * L, L)] = jnp.zeros((L,), jnp.float32)

    for k in range(2 * C // L):
        onesb[pl.ds(k * L, L)] = jnp.ones((L,), jnp.float32)

    z64 = yp0.at[pl.ds(0, C)]
    for j in range(9):
        pltpu.sync_copy(z64, agg_sp.at[pl.ds(sid * RPT + j * C, C)])
    pltpu.sync_copy(yp0.at[pl.ds(0, RPT - 9 * C)],
                    agg_sp.at[pl.ds(sid * RPT + 9 * C, RPT - 9 * C)])
    pltpu.sync_copy(zflat, cnt_sp.at[pl.ds(sid * (CNT_SZ // NS), CNT_SZ // NS)])

    plsc.subcore_barrier()

    n1 = jnp.minimum(jnp.maximum(EROWS - sid * NCH1, 0), NCH1)

    def _hist_idx_dma(c, s):
        pltpu.async_copy(edc_hbm.at[sid * NCH1 + c], c1b[s], hsem[s])

    for s in range(4):
        _hist_idx_dma(s, s)

    @pl.loop(0, n1, step=4)
    def _(i):
        for b in range(4):
            c = i + b

            @pl.when(c < n1)
            def _():
                @pl.when(c >= 4)
                def _():
                    pltpu.make_async_copy(onesb, cnt_sp.at[c1b[b]], hssem[b]).wait()

                pltpu.make_async_copy(edc_hbm.at[0], c1b[b], hsem[b]).wait()
                pltpu.async_copy(onesb, cnt_sp.at[c1b[b]], hssem[b], add=True)

                @pl.when(c + 4 < n1)
                def _():
                    _hist_idx_dma(c + 4, b)

    for b in range(4):
        @pl.when(b < n1)
        def _():
            pltpu.make_async_copy(onesb, cnt_sp.at[c1b[b]], hssem[b]).wait()

    plsc.subcore_barrier()

    wid = cid * NS + sid
    n2 = jnp.minimum(jnp.maximum(NREAL - wid * NCH2, 0), NCH2)

    def _main_idx_dma(c, s):
        g = wid * NCH2 + c
        pltpu.async_copy(eda_hbm.at[g], gpb[s], isem[s])
        pltpu.async_copy(edb_hbm.at[g], ebuf[s], isem[s])

    def _main_prep(c, s):
        pltpu.make_async_copy(eda_hbm.at[0], gpb[s], isem[s]).wait()
        pltpu.make_async_copy(edb_hbm.at[0], ebuf[s], isem[s]).wait()
        for k in range(C // L):
            sl = pl.ds(k * L, L)
            dstb[s][sl] = ebuf[s][1, sl]
        pltpu.async_copy(cnt_sp.at[ebuf[s].at[0]], cvb[s], csem[s])
        pltpu.async_copy(y_hbm.at[gpb[s]], ypair[s], ysem[s])

    _main_idx_dma(0, 0)
    _main_prep(0, 0)

    @pl.when(1 < n2)
    def _():
        _main_idx_dma(1, 1)

    @pl.loop(0, n2, step=2)
    def _(i):
        for b in (0, 1):
            c = i + b

            @pl.when(c >= 1)
            def _():
                pltpu.make_async_copy(ypair[1 - b].at[pl.ds(0, C)],
                                      agg_sp.at[dstb[1 - b]], scsem[1 - b]).wait()

            @pl.when(c + 1 < n2)
            def _():
                _main_prep(c + 1, 1 - b)

            pltpu.make_async_copy(y_hbm.at[gpb[b]], ypair[b], ysem[b]).wait()
            pltpu.make_async_copy(cnt_sp.at[ebuf[b].at[0]], cvb[b], csem[b]).wait()
            for k in range(C // L):
                sl = pl.ds(k * L, L)
                wbuf[sl] = 1.0 / jnp.maximum(cvb[b][sl], 1.0)

            @pl.when(c + 2 < n2)
            def _():
                _main_idx_dma(c + 2, b)

            @pl.loop(0, C, unroll=2)
            def _(e):
                w = wbuf[pl.ds(e, L)][0]
                for k in range(O // L):
                    sl = pl.ds(k * L, L)
                    ypair[b][e, sl] = (ypair[b][2 * e, sl] + ypair[b][2 * e + 1, sl]) * w

            pltpu.async_copy(ypair[b].at[pl.ds(0, C)], agg_sp.at[dstb[b]],
                             scsem[b], add=True)

    pltpu.make_async_copy(ypair[1].at[pl.ds(0, C)], agg_sp.at[dstb[1]],
                          scsem[1]).wait()

    plsc.subcore_barrier()

    bounce = yp0.at[pl.ds(0, C)]
    for j in range(9):
        r0 = sid * RPT + j * C
        pltpu.sync_copy(agg_sp.at[pl.ds(r0, C)], bounce)
        pltpu.sync_copy(bounce, out_hbm.at[cid, pl.ds(r0, C)])
    r0 = sid * RPT + 9 * C
    tail = RPT - 9 * C
    pltpu.sync_copy(agg_sp.at[pl.ds(r0, tail)], yp0.at[pl.ds(0, tail)])
    pltpu.sync_copy(yp0.at[pl.ds(0, tail)], out_hbm.at[cid, pl.ds(r0, tail)])


@functools.partial(
    pl.kernel,
    out_type=jax.ShapeDtypeStruct((NC, AGG_ROWS, O), jnp.float32),
    mesh=plsc.VectorSubcoreMesh(core_axis_name="c", subcore_axis_name="s"),
    scratch_types=[
        pltpu.VMEM((2560,), jnp.float32),
        pltpu.VMEM((2 * C, O), jnp.float32),
        pltpu.VMEM((2 * C, O), jnp.float32),
        pltpu.VMEM((2 * C,), jnp.int32),
        pltpu.VMEM((2 * C,), jnp.int32),
        pltpu.VMEM((2, C), jnp.int32),
        pltpu.VMEM((2, C), jnp.int32),
        pltpu.VMEM((2 * C,), jnp.int32),
        pltpu.VMEM((2 * C,), jnp.int32),
        pltpu.VMEM((2 * C,), jnp.int32),
        pltpu.VMEM((2 * C,), jnp.int32),
        pltpu.VMEM((C,), jnp.int32),
        pltpu.VMEM((C,), jnp.int32),
        pltpu.VMEM((C,), jnp.float32),
        pltpu.VMEM((C,), jnp.float32),
        pltpu.VMEM((C + L,), jnp.float32),
        pltpu.VMEM((2 * C,), jnp.float32),
        pltpu.VMEM_SHARED((AGG_ROWS, O), jnp.float32),
        pltpu.VMEM_SHARED((CNT_SZ,), jnp.float32),
        pltpu.SemaphoreType.DMA,
        pltpu.SemaphoreType.DMA,
        pltpu.SemaphoreType.DMA,
        pltpu.SemaphoreType.DMA,
        pltpu.SemaphoreType.DMA,
        pltpu.SemaphoreType.DMA,
        pltpu.SemaphoreType.DMA,
        pltpu.SemaphoreType.DMA,
    ],
)
def _sc_kernel(y, eda, edb, edc, out, *scratch):
    _sc_body(y, eda, edb, edc, out, *scratch)


@jax.jit
def kernel(x, hyperedge_index, hyperedge_type, A, W_C, b_C):
    hei = hyperedge_index.astype(jnp.int32)
    het = hyperedge_type.astype(jnp.int32)

    a8 = A.reshape(T, S, D, O).reshape(T * S, D, O)
    y = _tc_y(x, a8)
    gp, ci, ds = _tc_prep(hei.reshape(2, EROWS, 2 * D), het.reshape(EROWS, D))
    eda = gp.reshape(NREAL, 2 * C)
    edb = jnp.stack([ci.reshape(NREAL, C), ds.reshape(NREAL, C)], axis=1)
    agg = _sc_kernel(y, eda, edb, ci)
    return _tc_combine(x, W_C.T, agg, b_C.reshape(1, O))
